# Initial kernel scaffold; baseline (speedup 1.0000x reference)
import jax
import jax.numpy as jnp
from jax.experimental import pallas as pl


def kernel(x_ids, edge_index, batch, node_embeddings, W1, b1, W2, b2, W_ih, W_hh, b_ih, b_hh, initial_hs):
    raise NotImplementedError("write your pallas kernel here")


# speedup vs baseline: 24.3714x; 24.3714x over previous
import functools

import jax
import jax.numpy as jnp
from jax import lax
from jax.experimental import pallas as pl
from jax.experimental.pallas import tpu as pltpu
from jax.experimental.pallas import tpu_sc as plsc

N_NODES = 10000
N_EDGES = 160000
D = 128
N_GRAPHS = 64

MB = 400
TCH = 1024
LANES = 16
NW = 4
EP = N_EDGES + 8 * N_GRAPHS + TCH


def _mm_scale_body(x_ref, w_ref, s_ref, o_ref):
    o_ref[...] = jnp.dot(x_ref[...], w_ref[...],
                         preferred_element_type=jnp.float32) * s_ref[...]


def _mm_scale(x, w, s):
    M, K = x.shape
    Nw = w.shape[1]
    return pl.pallas_call(
        _mm_scale_body,
        grid=(M // MB,),
        in_specs=[
            pl.BlockSpec((MB, K), lambda i: (i, 0)),
            pl.BlockSpec((K, Nw), lambda i: (0, 0)),
            pl.BlockSpec((MB, 1), lambda i: (i, 0)),
        ],
        out_specs=pl.BlockSpec((MB, Nw), lambda i: (i, 0)),
        out_shape=jax.ShapeDtypeStruct((M, Nw), jnp.float32),
    )(x, w, s)


def _gru_body(xr_hbm, xz_hbm, xn_hbm, st_hbm, ct_hbm, par_hbm, aux_hbm,
              s_hbm, plast_hbm, p0_hbm, pfin_hbm, lnz_hbm,
              bufr, bufz, bufn, st_v, ct_v, par_v, aux_v, stf_v, sti_v, sem):
    w = lax.axis_index("s") * 2 + lax.axis_index("c")

    @pl.when(w < NW)
    def _():
        base_g = w * LANES
        pltpu.sync_copy(st_hbm.at[pl.ds(base_g, LANES)], st_v)
        pltpu.sync_copy(ct_hbm.at[pl.ds(base_g, LANES)], ct_v)
        pltpu.sync_copy(par_hbm, par_v)
        pars = par_v[...]
        wr = pars[0]
        wz = pars[1]
        wn = pars[2]
        bhr = pars[3]
        bhz = pars[4]
        bhn = pars[5]
        h0 = pars[6]
        cts = ct_v[...]
        sts = st_v[...]
        pltpu.sync_copy(aux_hbm, aux_v)
        nch = aux_v[...][0]
        lane = lax.broadcasted_iota(jnp.int32, (LANES,), 0)
        laneoff = lane * TCH

        def chunk(c, carry):
            copies = []
            for l in range(LANES):
                base = sts[l] + c * TCH
                base = jnp.minimum(base, EP - TCH)
                base = pl.multiple_of(base, 8)
                copies.append(pltpu.async_copy(
                    xr_hbm.at[pl.ds(base, TCH)], bufr.at[pl.ds(l * TCH, TCH)], sem))
                copies.append(pltpu.async_copy(
                    xz_hbm.at[pl.ds(base, TCH)], bufz.at[pl.ds(l * TCH, TCH)], sem))
                copies.append(pltpu.async_copy(
                    xn_hbm.at[pl.ds(base, TCH)], bufn.at[pl.ds(l * TCH, TCH)], sem))
            for cp in copies:
                cp.wait()

            def step(t, carry2):
                h, s, lnz, pla, p0, pf = carry2
                tv = jnp.full((LANES,), t, jnp.int32)
                fidx = laneoff + tv
                xr = plsc.load_gather(bufr, [fidx])
                xz = plsc.load_gather(bufz, [fidx])
                xn = plsc.load_gather(bufn, [fidx])
                r = 1.0 / (1.0 + jnp.exp(-(xr + h * wr + bhr)))
                z = 1.0 / (1.0 + jnp.exp(-(xz + h * wz + bhz)))
                an = xn + r * (h * wn + bhn)
                e2 = jnp.exp(-2.0 * jnp.abs(an))
                nn = jnp.sign(an) * ((1.0 - e2) / (1.0 + e2))
                h2 = (1.0 - z) * nn + z * h
                tg = tv + c * TCH
                act = tg < cts
                nz = jnp.logical_and(act, h2 != 0.0)
                s = jnp.where(act, s + h2, s)
                lnz = jnp.where(nz, tg, lnz)
                pla = jnp.where(nz, h2, pla)
                p0 = jnp.where(jnp.logical_and(act, tg == 0), h2, p0)
                pf = jnp.where(act, h2, pf)
                h = jnp.where(act, h2, h)
                return h, s, lnz, pla, p0, pf

            return lax.fori_loop(0, TCH, step, carry)

        zf = jnp.zeros((LANES,), jnp.float32)
        carry0 = (jnp.full((LANES,), h0, jnp.float32), zf,
                  jnp.full((LANES,), -1, jnp.int32), zf, zf, zf)
        _, s, lnz, pla, p0, pf = lax.fori_loop(0, nch, chunk, carry0)

        stf_v[...] = s
        pltpu.sync_copy(stf_v, s_hbm.at[pl.ds(base_g, LANES)])
        stf_v[...] = pla
        pltpu.sync_copy(stf_v, plast_hbm.at[pl.ds(base_g, LANES)])
        stf_v[...] = p0
        pltpu.sync_copy(stf_v, p0_hbm.at[pl.ds(base_g, LANES)])
        stf_v[...] = pf
        pltpu.sync_copy(stf_v, pfin_hbm.at[pl.ds(base_g, LANES)])
        sti_v[...] = lnz
        pltpu.sync_copy(sti_v, lnz_hbm.at[pl.ds(base_g, LANES)])


_gru = functools.partial(
    pl.kernel,
    out_type=[jax.ShapeDtypeStruct((N_GRAPHS,), jnp.float32)] * 4
             + [jax.ShapeDtypeStruct((N_GRAPHS,), jnp.int32)],
    mesh=plsc.VectorSubcoreMesh(core_axis_name="c", subcore_axis_name="s"),
    compiler_params=pltpu.CompilerParams(needs_layout_passes=False),
    scratch_types=[
        pltpu.VMEM((LANES * TCH,), jnp.float32),
        pltpu.VMEM((LANES * TCH,), jnp.float32),
        pltpu.VMEM((LANES * TCH,), jnp.float32),
        pltpu.VMEM((LANES,), jnp.int32),
        pltpu.VMEM((LANES,), jnp.int32),
        pltpu.VMEM((16,), jnp.float32),
        pltpu.VMEM((16,), jnp.int32),
        pltpu.VMEM((LANES,), jnp.float32),
        pltpu.VMEM((LANES,), jnp.int32),
        pltpu.SemaphoreType.DMA,
    ],
)(_gru_body)


def kernel(x_ids, edge_index, batch, node_embeddings, W1, b1, W2, b2,
           W_ih, W_hh, b_ih, b_hh, initial_hs):
    row, col = edge_index[0], edge_index[1]
    x = node_embeddings[x_ids]
    deg = jnp.zeros((N_NODES,), jnp.float32).at[col].add(1.0) + 1.0
    dinv = lax.rsqrt(deg)
    dcol = dinv[:, None]

    def gcn(x, W, b):
        xs = _mm_scale(x, W, dcol)
        agg = jnp.zeros((N_NODES, D), jnp.float32).at[col].add(xs[row])
        return dcol * (agg + xs) + b

    x = jax.nn.gelu(gcn(x, W1, b1), approximate=False)
    x = jax.nn.gelu(gcn(x, W2, b2), approximate=False)

    W3 = jnp.zeros((D, D), jnp.float32).at[:, :3].set(W_ih.T)
    x3 = _mm_scale(x, W3, jnp.ones((N_NODES, 1), jnp.float32))[:, :3]

    ge = batch[row]
    xg = x3[row] + x3[col] + b_ih
    counts = jnp.bincount(ge, length=N_GRAPHS).astype(jnp.int32)
    cnt8 = ((counts + 7) // 8) * 8
    start_al = (jnp.cumsum(cnt8) - cnt8).astype(jnp.int32)
    start_pk = jnp.cumsum(counts) - counts
    perm = jnp.argsort(ge, stable=True)
    gs = ge[perm]
    rank = jnp.arange(N_EDGES, dtype=jnp.int32) - start_pk[gs]
    pos = start_al[gs] + rank
    xgp = xg[perm]
    planes = [jnp.zeros((EP,), jnp.float32).at[pos].set(xgp[:, j])
              for j in range(3)]
    params = jnp.concatenate([W_hh[:, 0], b_hh, initial_hs[0],
                              jnp.zeros((9,), jnp.float32)])
    nch = (jnp.max(counts) + TCH - 1) // TCH
    aux = jnp.zeros((16,), jnp.int32).at[0].set(nch)
    s, pla, p0, pf, lnz = _gru(planes[0], planes[1], planes[2],
                               start_al, counts, params, aux)

    idx = jnp.arange(N_GRAPHS)
    has = counts > 0
    nxt = lax.cummin(jnp.where(has, idx, 2 * N_GRAPHS), axis=0, reverse=True)
    last_ne = jnp.max(jnp.where(has, idx, -1))
    fallback = pf[jnp.maximum(last_ne, 0)]
    base_pred = jnp.where(nxt < N_GRAPHS,
                          p0[jnp.clip(nxt, 0, N_GRAPHS - 1)], fallback)
    cond = (s > 0) & (lnz >= 0)
    return jnp.where(cond, pla, base_pred)
